# baseline (speedup 1.0000x reference)
import jax
import jax.numpy as jnp
from jax import lax
from jax.experimental import pallas as pl
from jax.experimental.pallas import tpu as pltpu
from jax.experimental.pallas import tpu_sc as plsc

NUM_ROWS = 100000
DIM = 64
BATCH = 4096
HIST = 200

_info = plsc.get_sparse_core_info()
NC, NS = _info.num_cores, _info.num_subcores
NW = NC * NS
BTILE = BATCH // NW
DT = DIM // 8


def _add_body(a_ref, b_ref, o_ref):
    o_ref[...] = a_ref[...] + b_ref[...]


def _summed_table(emb, sin):
    rows_blk = 4000
    grid = NUM_ROWS // rows_blk
    spec = pl.BlockSpec((rows_blk, DIM), lambda i: (i, 0))
    return pl.pallas_call(
        _add_body,
        grid=(grid,),
        in_specs=[spec, spec],
        out_specs=spec,
        out_shape=jax.ShapeDtypeStruct((NUM_ROWS, DIM), jnp.float32),
    )(emb, sin)


def _gather_body(table_hbm, idx_hbm, y_hbm, idx_v, grows_v, tbuf_v,
                 g0, g1, w0, w1):
    wid = lax.axis_index("s") * NC + lax.axis_index("c")
    gsem = (g0, g1)
    wsem = (w0, w1)
    pltpu.sync_copy(idx_hbm.at[wid], idx_v)

    lanes = lax.iota(jnp.int32, 16)

    def fire_gather(t, b):
        pltpu.async_copy(table_hbm.at[idx_v.at[t]], grows_v.at[b], gsem[b])

    def wait_gather(b):
        pltpu.make_async_copy(
            table_hbm.at[pl.ds(0, BTILE)], grows_v.at[b], gsem[b]
        ).wait()

    def transpose(b):
        src = grows_v.at[b]
        dst = tbuf_v.at[b]

        def dt_step(dt, _):
            for dr in range(8):
                d = dt * 8 + dr
                dcol = jnp.full((16,), d, jnp.int32)
                for g in range(8):
                    v = plsc.load_gather(src, [lanes + g * 16, dcol])
                    dst[pl.ds(dt * 1024 + dr * 128 + g * 16, 16)] = v
            return 0

        lax.fori_loop(0, DT, dt_step, 0)

    def fire_wb(t, b):
        for dt in range(DT):
            pltpu.async_copy(
                tbuf_v.at[b].at[pl.ds(dt * 1024, 1024)],
                y_hbm.at[t].at[dt].at[wid],
                wsem[b],
            )

    def wait_wb(b):
        pltpu.make_async_copy(
            table_hbm.at[pl.ds(0, BTILE)], grows_v.at[b], wsem[b]
        ).wait()

    fire_gather(0, 0)
    for t in (0, 1):
        b = t % 2
        fire_gather(t + 1, (t + 1) % 2)
        wait_gather(b)
        transpose(b)
        fire_wb(t, b)

    def loop(t2, _):
        for k in range(2):
            t = 2 + 2 * t2 + k
            b = k
            fire_gather(t + 1, 1 - b)
            wait_gather(b)
            wait_wb(b)
            transpose(b)
            fire_wb(t, b)
        return 0

    lax.fori_loop(0, (HIST - 4) // 2, loop, 0)

    for t in (HIST - 2, HIST - 1):
        b = t % 2
        if t == HIST - 2:
            fire_gather(t + 1, (t + 1) % 2)
        wait_gather(b)
        wait_wb(b)
        transpose(b)
        fire_wb(t, b)
    wait_wb(0)
    wait_wb(1)


_gather = pl.kernel(
    _gather_body,
    out_type=jax.ShapeDtypeStruct((HIST, DT, NW, 1024), jnp.float32),
    mesh=plsc.VectorSubcoreMesh(core_axis_name="c", subcore_axis_name="s"),
    scratch_types=[
        pltpu.VMEM((HIST, BTILE), jnp.int32),
        pltpu.VMEM((2, BTILE, DIM), jnp.float32),
        pltpu.VMEM((2, DT * 1024), jnp.float32),
        pltpu.SemaphoreType.DMA,
        pltpu.SemaphoreType.DMA,
        pltpu.SemaphoreType.DMA,
        pltpu.SemaphoreType.DMA,
    ],
    compiler_params=pltpu.CompilerParams(
        use_tc_tiling_on_sc=False, needs_layout_passes=False
    ),
)


def kernel(index_tensor, embedding_matrix, sinusoid_table):
    table = _summed_table(embedding_matrix, sinusoid_table)
    idx = (
        index_tensor.astype(jnp.int32)
        .reshape(NW, BTILE, HIST)
        .transpose(0, 2, 1)
    )
    y = _gather(table, idx)
    y = y.reshape(HIST, DT, NW, 8, 128)
    return y.transpose(2, 4, 0, 1, 3).reshape(BATCH, HIST, DIM)


# speedup vs baseline: 2.5526x; 2.5526x over previous
import jax
import jax.numpy as jnp
from jax import lax
from jax.experimental import pallas as pl
from jax.experimental.pallas import tpu as pltpu
from jax.experimental.pallas import tpu_sc as plsc

NUM_ROWS = 100000
DIM = 64
BATCH = 4096
HIST = 200

_info = plsc.get_sparse_core_info()
NC, NS = _info.num_cores, _info.num_subcores
NW = NC * NS
BTILE = BATCH // NW
DT = DIM // 8
PITCH = 129


def _add_body(a_ref, b_ref, o_ref):
    o_ref[...] = a_ref[...] + b_ref[...]


def _summed_table(emb, sin):
    rows_blk = 4000
    grid = NUM_ROWS // rows_blk
    spec = pl.BlockSpec((rows_blk, DIM), lambda i: (i, 0))
    return pl.pallas_call(
        _add_body,
        grid=(grid,),
        in_specs=[spec, spec],
        out_specs=spec,
        out_shape=jax.ShapeDtypeStruct((NUM_ROWS, DIM), jnp.float32),
    )(emb, sin)


def _gather_body(table_hbm, idx_hbm, y_hbm, idx_v, grows_v, tbuf_v,
                 g0, g1, w0, w1):
    wid = lax.axis_index("s") * NC + lax.axis_index("c")
    gsem = (g0, g1)
    wsem = (w0, w1)
    pltpu.sync_copy(idx_hbm.at[wid], idx_v)

    lanes = lax.iota(jnp.int32, 16)

    def fire_gather(t, b):
        pltpu.async_copy(table_hbm.at[idx_v.at[t]], grows_v.at[b], gsem[b])

    def wait_gather(b):
        pltpu.make_async_copy(
            table_hbm.at[pl.ds(0, BTILE)], grows_v.at[b], gsem[b]
        ).wait()

    def transpose(b):
        src = grows_v.at[b]
        dst = tbuf_v.at[b]

        def j_step(j8, _):
            for jr in range(8):
                j = j8 * 8 + jr
                jcol = jnp.full((16,), j, jnp.int32)
                for k in range(4):
                    v = src[j, pl.ds(k * 16, 16)]
                    plsc.store_scatter(dst, [lanes + k * 16, jcol], v)
            return 0

        lax.fori_loop(0, BTILE // 8, j_step, 0)

    def fire_wb(t, b):
        for dt in range(DT):
            pltpu.async_copy(
                tbuf_v.at[b].at[pl.ds(dt * 8, 8), pl.ds(0, 128)],
                y_hbm.at[t].at[dt].at[wid],
                wsem[b],
            )

    def wait_wb(b):
        pltpu.make_async_copy(
            table_hbm.at[pl.ds(0, BTILE)], grows_v.at[b], wsem[b]
        ).wait()

    fire_gather(0, 0)
    for t in (0, 1):
        b = t % 2
        fire_gather(t + 1, (t + 1) % 2)
        wait_gather(b)
        transpose(b)
        fire_wb(t, b)

    def loop(t2, _):
        for k in range(2):
            t = 2 + 2 * t2 + k
            b = k
            fire_gather(t + 1, 1 - b)
            wait_gather(b)
            wait_wb(b)
            transpose(b)
            fire_wb(t, b)
        return 0

    lax.fori_loop(0, (HIST - 4) // 2, loop, 0)

    for t in (HIST - 2, HIST - 1):
        b = t % 2
        if t == HIST - 2:
            fire_gather(t + 1, (t + 1) % 2)
        wait_gather(b)
        wait_wb(b)
        transpose(b)
        fire_wb(t, b)
    wait_wb(0)
    wait_wb(1)


_gather = pl.kernel(
    _gather_body,
    out_type=jax.ShapeDtypeStruct((HIST, DT, NW, 8, 128), jnp.float32),
    mesh=plsc.VectorSubcoreMesh(core_axis_name="c", subcore_axis_name="s"),
    scratch_types=[
        pltpu.VMEM((HIST, BTILE), jnp.int32),
        pltpu.VMEM((2, BTILE, DIM), jnp.float32),
        pltpu.VMEM((2, DIM, PITCH), jnp.float32),
        pltpu.SemaphoreType.DMA,
        pltpu.SemaphoreType.DMA,
        pltpu.SemaphoreType.DMA,
        pltpu.SemaphoreType.DMA,
    ],
    compiler_params=pltpu.CompilerParams(
        use_tc_tiling_on_sc=False, needs_layout_passes=False
    ),
)


def kernel(index_tensor, embedding_matrix, sinusoid_table):
    table = _summed_table(embedding_matrix, sinusoid_table)
    idx = (
        index_tensor.astype(jnp.int32)
        .reshape(NW, BTILE, HIST)
        .transpose(0, 2, 1)
    )
    y = _gather(table, idx)
    return y.transpose(2, 4, 0, 1, 3).reshape(BATCH, HIST, DIM)


# speedup vs baseline: 2.9951x; 1.1733x over previous
import jax
import jax.numpy as jnp
from jax import lax
from jax.experimental import pallas as pl
from jax.experimental.pallas import tpu as pltpu
from jax.experimental.pallas import tpu_sc as plsc

NUM_ROWS = 100000
DIM = 64
BATCH = 4096
HIST = 200

_info = plsc.get_sparse_core_info()
NC, NS = _info.num_cores, _info.num_subcores
NW = NC * NS
BTILE = BATCH // NW
DT = DIM // 8
PITCH = 129


def _add_t_body(a_ref, b_ref, o_ref):
    o_ref[...] = (a_ref[...] + b_ref[...]).T


def _summed_table(emb_t, sin_t):
    rows_blk = 4096
    grid = pl.cdiv(NUM_ROWS, rows_blk)
    in_spec = pl.BlockSpec((DIM, rows_blk), lambda i: (0, i))
    out_spec = pl.BlockSpec((rows_blk, DIM), lambda i: (i, 0))
    return pl.pallas_call(
        _add_t_body,
        grid=(grid,),
        in_specs=[in_spec, in_spec],
        out_specs=out_spec,
        out_shape=jax.ShapeDtypeStruct((NUM_ROWS, DIM), jnp.float32),
    )(emb_t, sin_t)


def _gather_body(table_hbm, idx_hbm, y_hbm, idx_v, grows_v, tbuf_v,
                 g0, g1, w0, w1):
    wid = lax.axis_index("s") * NC + lax.axis_index("c")
    gsem = (g0, g1)
    wsem = (w0, w1)
    pltpu.sync_copy(idx_hbm.at[wid], idx_v)

    lanes = lax.iota(jnp.int32, 16)
    dtv = [(lanes + k * 16) >> 3 for k in range(4)]
    drv = [(lanes + k * 16) & 7 for k in range(4)]

    def fire_gather(t, b):
        pltpu.async_copy(table_hbm.at[idx_v.at[t]], grows_v.at[b], gsem[b])

    def wait_gather(b):
        pltpu.make_async_copy(
            table_hbm.at[pl.ds(0, BTILE)], grows_v.at[b], gsem[b]
        ).wait()

    def transpose(b):
        src = grows_v.at[b]
        dst = tbuf_v.at[b]

        def j_step(j8, _):
            for jr in range(8):
                j = j8 * 8 + jr
                jcol = jnp.full((16,), j, jnp.int32)
                for k in range(4):
                    v = src[j, pl.ds(k * 16, 16)]
                    plsc.store_scatter(dst, [dtv[k], drv[k], jcol], v)
            return 0

        lax.fori_loop(0, BTILE // 8, j_step, 0)

    def fire_wb(t, b):
        pltpu.async_copy(
            tbuf_v.at[b].at[:, :, pl.ds(0, 128)],
            y_hbm.at[t].at[:, wid],
            wsem[b],
        )

    def wait_wb(b):
        pltpu.make_async_copy(
            table_hbm.at[pl.ds(0, BTILE)], grows_v.at[b], wsem[b]
        ).wait()

    fire_gather(0, 0)
    for t in (0, 1):
        b = t % 2
        fire_gather(t + 1, (t + 1) % 2)
        wait_gather(b)
        transpose(b)
        fire_wb(t, b)

    def loop(t2, _):
        for k in range(2):
            t = 2 + 2 * t2 + k
            b = k
            fire_gather(t + 1, 1 - b)
            wait_gather(b)
            wait_wb(b)
            transpose(b)
            fire_wb(t, b)
        return 0

    lax.fori_loop(0, (HIST - 4) // 2, loop, 0)

    for t in (HIST - 2, HIST - 1):
        b = t % 2
        if t == HIST - 2:
            fire_gather(t + 1, (t + 1) % 2)
        wait_gather(b)
        wait_wb(b)
        transpose(b)
        fire_wb(t, b)
    wait_wb(0)
    wait_wb(1)


_gather = pl.kernel(
    _gather_body,
    out_type=jax.ShapeDtypeStruct((HIST, DT, NW, 8, 128), jnp.float32),
    mesh=plsc.VectorSubcoreMesh(core_axis_name="c", subcore_axis_name="s"),
    scratch_types=[
        pltpu.VMEM((HIST, BTILE), jnp.int32),
        pltpu.VMEM((2, BTILE, DIM), jnp.float32),
        pltpu.VMEM((2, DT, 8, PITCH), jnp.float32),
        pltpu.SemaphoreType.DMA,
        pltpu.SemaphoreType.DMA,
        pltpu.SemaphoreType.DMA,
        pltpu.SemaphoreType.DMA,
    ],
    compiler_params=pltpu.CompilerParams(
        use_tc_tiling_on_sc=False, needs_layout_passes=False
    ),
)


def kernel(index_tensor, embedding_matrix, sinusoid_table):
    table = _summed_table(embedding_matrix.T, sinusoid_table.T)
    idx = (
        index_tensor.astype(jnp.int32)
        .reshape(NW, BTILE, HIST)
        .transpose(0, 2, 1)
    )
    y = _gather(table, idx)
    return y.transpose(2, 4, 0, 1, 3).reshape(BATCH, HIST, DIM)
